# Initial kernel scaffold; baseline (speedup 1.0000x reference)
import jax
import jax.numpy as jnp
from jax.experimental import pallas as pl


def kernel(indices, num_x, tables):
    raise NotImplementedError("write your pallas kernel here")


# speedup vs baseline: 3.0127x; 3.0127x over previous
import functools

import jax
import jax.numpy as jnp
from jax import lax
from jax.experimental import pallas as pl
from jax.experimental.pallas import tpu as pltpu
from jax.experimental.pallas import tpu_sc as plsc

N_FIELDS = 26
VOCAB = 100000
DIM = 16
BATCH = 4096
HIST = 20

NC = 2
NS = 16
NW = NC * NS
NB = BATCH // NW
ROWS = NB * HIST
CHUNK = 128
NCHUNK = ROWS // CHUNK


def _sc_call(idx_flat, tables_flat):
    mesh = plsc.VectorSubcoreMesh(
        core_axis_name="c", subcore_axis_name="s", num_cores=NC, num_subcores=NS
    )

    @functools.partial(
        pl.kernel,
        out_type=jax.ShapeDtypeStruct((BATCH, N_FIELDS * DIM), jnp.float32),
        mesh=mesh,
        scratch_types=[
            pltpu.VMEM((ROWS,), jnp.int32),
            pltpu.VMEM((NCHUNK, CHUNK), jnp.int32),
            pltpu.VMEM((ROWS, DIM), jnp.float32),
            pltpu.VMEM((NB, N_FIELDS * DIM), jnp.float32),
            pltpu.SemaphoreType.DMA,
        ],
        compiler_params=pltpu.CompilerParams(use_tc_tiling_on_sc=False),
    )
    def k(idx_hbm, tab_hbm, out_hbm, idx_v, gidx_v, gbuf, oblk, sem):
        wid = lax.axis_index("s") * NC + lax.axis_index("c")
        base = wid * NB

        @pl.loop(0, N_FIELDS)
        def per_field(f):
            off = f * (BATCH * HIST) + base * HIST
            pltpu.sync_copy(idx_hbm.at[pl.ds(off, ROWS)], idx_v)
            fbase = f * VOCAB
            for c in range(NCHUNK):
                for j in range(CHUNK // 16):
                    v = idx_v[pl.ds(c * CHUNK + j * 16, 16)]
                    gidx_v[c, pl.ds(j * 16, 16)] = v + fbase
            copies = [
                pltpu.async_copy(
                    tab_hbm.at[gidx_v.at[c]],
                    gbuf.at[pl.ds(c * CHUNK, CHUNK)],
                    sem,
                )
                for c in range(NCHUNK)
            ]
            for cp in copies:
                cp.wait()
            col = f * DIM

            @pl.loop(0, NB)
            def per_b(b):
                r0 = b * HIST
                v = gbuf[r0, :]
                for h in range(1, HIST):
                    v = v + gbuf[r0 + h, :]
                oblk[b, pl.ds(col, DIM)] = v * (1.0 / HIST)

        pltpu.sync_copy(oblk, out_hbm.at[pl.ds(base, NB)])

    return k(idx_flat, tables_flat)


def kernel(indices, num_x, tables):
    idx_flat = indices.astype(jnp.int32).reshape(-1)
    tables_flat = tables.reshape(N_FIELDS * VOCAB, DIM)
    emb = _sc_call(idx_flat, tables_flat)
    return jnp.concatenate([emb, num_x.astype(jnp.float32)], axis=1)
